# baseline (device time: 131906 ns/iter reference)
import jax
import jax.numpy as jnp
from jax import lax
from jax.experimental import pallas as pl
from jax.experimental.pallas import tpu as pltpu

C = 16


def kernel(x):
    _, m_total, n2 = x.shape
    n = n2 // 2
    half_m = m_total // 2
    rc = half_m // C

    def body(x_hbm, out_ref, f32_buf, send_buf, recv_buf,
             copy_sems, xs_sems, xr_sems, ys_sems, yr_sems):
        mx = lax.axis_index("x")
        my = lax.axis_index("y")
        px = 1 - mx
        py = 1 - my
        row0 = my * half_m

        def local_dma(c):
            return pltpu.make_async_copy(
                x_hbm.at[0, pl.ds(row0 + c * rc, rc), :],
                f32_buf.at[c % 2],
                copy_sems.at[c % 2],
            )

        local_dma(0).start()
        x_rdmas = []
        for c in range(C):
            if c + 1 < C:
                local_dma(c + 1).start()
            local_dma(c).wait()
            rs = row0 + c * rc

            @pl.when(mx == 0)
            def _():
                out_ref[pl.ds(rs, rc), :] = (
                    f32_buf[c % 2, :, :n].astype(jnp.bfloat16)
                )
                send_buf[c] = f32_buf[c % 2, :, n:].astype(jnp.bfloat16)

            @pl.when(mx == 1)
            def _():
                out_ref[pl.ds(rs, rc), :] = (
                    f32_buf[c % 2, :, n:].astype(jnp.bfloat16)
                )
                send_buf[c] = f32_buf[c % 2, :, :n].astype(jnp.bfloat16)
            rdma = pltpu.make_async_remote_copy(
                src_ref=send_buf.at[c],
                dst_ref=recv_buf.at[c],
                send_sem=xs_sems.at[c],
                recv_sem=xr_sems.at[c],
                device_id=(px, my),
                device_id_type=pl.DeviceIdType.MESH,
            )
            rdma.start()
            x_rdmas.append(rdma)

        y_rdmas = []
        for c in range(C):
            x_rdmas[c].wait_recv()
            rs = row0 + c * rc
            out_ref[pl.ds(rs, rc), :] = (
                out_ref[pl.ds(rs, rc), :] + recv_buf[c]
            )
            rdma = pltpu.make_async_remote_copy(
                src_ref=out_ref.at[pl.ds(rs, rc), :],
                dst_ref=out_ref.at[pl.ds(rs, rc), :],
                send_sem=ys_sems.at[c],
                recv_sem=yr_sems.at[c],
                device_id=(mx, py),
                device_id_type=pl.DeviceIdType.MESH,
            )
            rdma.start()
            y_rdmas.append(rdma)

        for c in range(C):
            y_rdmas[c].wait_recv()
        for c in range(C):
            x_rdmas[c].wait_send()
            y_rdmas[c].wait_send()

    return pl.pallas_call(
        body,
        out_shape=jax.ShapeDtypeStruct((m_total, n), jnp.bfloat16),
        in_specs=[pl.BlockSpec(memory_space=pltpu.HBM)],
        out_specs=pl.BlockSpec(memory_space=pltpu.VMEM),
        scratch_shapes=[
            pltpu.VMEM((2, rc, n2), jnp.float32),
            pltpu.VMEM((C, rc, n), jnp.bfloat16),
            pltpu.VMEM((C, rc, n), jnp.bfloat16),
            pltpu.SemaphoreType.DMA((2,)),
            pltpu.SemaphoreType.DMA((C,)),
            pltpu.SemaphoreType.DMA((C,)),
            pltpu.SemaphoreType.DMA((C,)),
            pltpu.SemaphoreType.DMA((C,)),
        ],
        compiler_params=pltpu.CompilerParams(
            vmem_limit_bytes=56 * 1024 * 1024,
        ),
    )(x)


# device time: 122789 ns/iter; 1.0742x vs baseline; 1.0742x over previous
import os

import jax
import jax.numpy as jnp
from jax import lax
from jax.experimental import pallas as pl
from jax.experimental.pallas import tpu as pltpu

C = int(os.environ.get("RS_CHUNKS", "16"))


def kernel(x):
    _, m_total, n2 = x.shape
    n = n2 // 2
    half_m = m_total // 2
    rc = half_m // C

    def body(x_hbm, out_hbm, f32_buf, send_buf, recv_buf, sum_buf,
             in_sems, out_sems, xs_sems, xr_sems, ys_sems, yr_sems):
        mx = lax.axis_index("x")
        my = lax.axis_index("y")
        px = 1 - mx
        py = 1 - my
        row0 = my * half_m

        barrier_sem = pltpu.get_barrier_semaphore()
        pl.semaphore_signal(barrier_sem, inc=1, device_id=(px, my),
                            device_id_type=pl.DeviceIdType.MESH)
        pl.semaphore_signal(barrier_sem, inc=1, device_id=(mx, py),
                            device_id_type=pl.DeviceIdType.MESH)
        pl.semaphore_wait(barrier_sem, 2)

        def local_in_dma(c):
            return pltpu.make_async_copy(
                x_hbm.at[0, pl.ds(row0 + c * rc, rc), :],
                f32_buf.at[c % 2],
                in_sems.at[c % 2],
            )

        local_in_dma(0).start()
        x_rdmas = []
        for c in range(C):
            if c + 1 < C:
                local_in_dma(c + 1).start()
            local_in_dma(c).wait()

            @pl.when(mx == 0)
            def _():
                sum_buf[c] = f32_buf[c % 2, :, :n].astype(jnp.bfloat16)
                send_buf[c] = f32_buf[c % 2, :, n:].astype(jnp.bfloat16)

            @pl.when(mx == 1)
            def _():
                sum_buf[c] = f32_buf[c % 2, :, n:].astype(jnp.bfloat16)
                send_buf[c] = f32_buf[c % 2, :, :n].astype(jnp.bfloat16)

            rdma = pltpu.make_async_remote_copy(
                src_ref=send_buf.at[c],
                dst_ref=recv_buf.at[c],
                send_sem=xs_sems.at[c],
                recv_sem=xr_sems.at[c],
                device_id=(px, my),
                device_id_type=pl.DeviceIdType.MESH,
            )
            rdma.start()
            x_rdmas.append(rdma)

        y_rdmas = []
        out_dmas = []
        for c in range(C):
            x_rdmas[c].wait_recv()
            sum_buf[c] = sum_buf[c] + recv_buf[c]
            rs = row0 + c * rc
            out_dma = pltpu.make_async_copy(
                sum_buf.at[c],
                out_hbm.at[pl.ds(rs, rc), :],
                out_sems.at[c],
            )
            out_dma.start()
            out_dmas.append(out_dma)
            rdma = pltpu.make_async_remote_copy(
                src_ref=sum_buf.at[c],
                dst_ref=out_hbm.at[pl.ds(rs, rc), :],
                send_sem=ys_sems.at[c],
                recv_sem=yr_sems.at[c],
                device_id=(mx, py),
                device_id_type=pl.DeviceIdType.MESH,
            )
            rdma.start()
            y_rdmas.append(rdma)

        for c in range(C):
            y_rdmas[c].wait_recv()
            out_dmas[c].wait()
        for c in range(C):
            x_rdmas[c].wait_send()
            y_rdmas[c].wait_send()

    return pl.pallas_call(
        body,
        out_shape=jax.ShapeDtypeStruct((m_total, n), jnp.bfloat16),
        in_specs=[pl.BlockSpec(memory_space=pltpu.HBM)],
        out_specs=pl.BlockSpec(memory_space=pltpu.HBM),
        scratch_shapes=[
            pltpu.VMEM((2, rc, n2), jnp.float32),
            pltpu.VMEM((C, rc, n), jnp.bfloat16),
            pltpu.VMEM((C, rc, n), jnp.bfloat16),
            pltpu.VMEM((C, rc, n), jnp.bfloat16),
            pltpu.SemaphoreType.DMA((2,)),
            pltpu.SemaphoreType.DMA((C,)),
            pltpu.SemaphoreType.DMA((C,)),
            pltpu.SemaphoreType.DMA((C,)),
            pltpu.SemaphoreType.DMA((C,)),
            pltpu.SemaphoreType.DMA((C,)),
        ],
        compiler_params=pltpu.CompilerParams(
            vmem_limit_bytes=56 * 1024 * 1024,
            collective_id=0,
        ),
    )(x)


# device time: 122507 ns/iter; 1.0767x vs baseline; 1.0023x over previous
import jax
import jax.numpy as jnp
from jax import lax
from jax.experimental import pallas as pl
from jax.experimental.pallas import tpu as pltpu

SIZES = [64, 128, 256, 512, 512, 512, 512, 512, 512, 320, 128, 64, 32, 32]
OFFS = [sum(SIZES[:i]) for i in range(len(SIZES))]
C = len(SIZES)


def kernel(x):
    _, m_total, n2 = x.shape
    n = n2 // 2
    half_m = m_total // 2
    assert sum(SIZES) == half_m
    max_rc = max(SIZES)

    def body(x_hbm, out_hbm, f32_buf, send_buf, recv_buf, sum_buf,
             in_sems, out_sems, xs_sems, xr_sems, ys_sems, yr_sems):
        mx = lax.axis_index("x")
        my = lax.axis_index("y")
        px = 1 - mx
        py = 1 - my
        row0 = my * half_m

        barrier_sem = pltpu.get_barrier_semaphore()
        pl.semaphore_signal(barrier_sem, inc=1, device_id=(px, my),
                            device_id_type=pl.DeviceIdType.MESH)
        pl.semaphore_signal(barrier_sem, inc=1, device_id=(mx, py),
                            device_id_type=pl.DeviceIdType.MESH)

        def local_in_dma(c):
            return pltpu.make_async_copy(
                x_hbm.at[0, pl.ds(row0 + OFFS[c], SIZES[c]), :],
                f32_buf.at[c % 2, : SIZES[c]],
                in_sems.at[c % 2],
            )

        local_in_dma(0).start()
        x_rdmas = []
        for c in range(C):
            sz, off = SIZES[c], OFFS[c]
            sl = slice(off, off + sz)
            if c + 1 < C:
                local_in_dma(c + 1).start()
            local_in_dma(c).wait()

            @pl.when(mx == 0)
            def _():
                sum_buf[sl] = f32_buf[c % 2, :sz, :n].astype(jnp.bfloat16)
                send_buf[sl] = f32_buf[c % 2, :sz, n:].astype(jnp.bfloat16)

            @pl.when(mx == 1)
            def _():
                sum_buf[sl] = f32_buf[c % 2, :sz, n:].astype(jnp.bfloat16)
                send_buf[sl] = f32_buf[c % 2, :sz, :n].astype(jnp.bfloat16)

            if c == 0:
                pl.semaphore_wait(barrier_sem, 2)
            rdma = pltpu.make_async_remote_copy(
                src_ref=send_buf.at[sl],
                dst_ref=recv_buf.at[sl],
                send_sem=xs_sems.at[c],
                recv_sem=xr_sems.at[c],
                device_id=(px, my),
                device_id_type=pl.DeviceIdType.MESH,
            )
            rdma.start()
            x_rdmas.append(rdma)

        y_rdmas = []
        out_dmas = []
        for c in range(C):
            sz, off = SIZES[c], OFFS[c]
            sl = slice(off, off + sz)
            x_rdmas[c].wait_recv()
            sum_buf[sl] = sum_buf[sl] + recv_buf[sl]
            out_slice = out_hbm.at[pl.ds(row0 + off, sz), :]
            out_dma = pltpu.make_async_copy(
                sum_buf.at[sl], out_slice, out_sems.at[c]
            )
            out_dma.start()
            out_dmas.append(out_dma)
            rdma = pltpu.make_async_remote_copy(
                src_ref=sum_buf.at[sl],
                dst_ref=out_slice,
                send_sem=ys_sems.at[c],
                recv_sem=yr_sems.at[c],
                device_id=(mx, py),
                device_id_type=pl.DeviceIdType.MESH,
            )
            rdma.start()
            y_rdmas.append(rdma)

        for c in range(C):
            y_rdmas[c].wait_recv()
            out_dmas[c].wait()
        for c in range(C):
            x_rdmas[c].wait_send()
            y_rdmas[c].wait_send()

    return pl.pallas_call(
        body,
        out_shape=jax.ShapeDtypeStruct((m_total, n), jnp.bfloat16),
        in_specs=[pl.BlockSpec(memory_space=pltpu.HBM)],
        out_specs=pl.BlockSpec(memory_space=pltpu.HBM),
        scratch_shapes=[
            pltpu.VMEM((2, max_rc, n2), jnp.float32),
            pltpu.VMEM((half_m, n), jnp.bfloat16),
            pltpu.VMEM((half_m, n), jnp.bfloat16),
            pltpu.VMEM((half_m, n), jnp.bfloat16),
            pltpu.SemaphoreType.DMA((2,)),
            pltpu.SemaphoreType.DMA((C,)),
            pltpu.SemaphoreType.DMA((C,)),
            pltpu.SemaphoreType.DMA((C,)),
            pltpu.SemaphoreType.DMA((C,)),
            pltpu.SemaphoreType.DMA((C,)),
        ],
        compiler_params=pltpu.CompilerParams(
            vmem_limit_bytes=56 * 1024 * 1024,
            collective_id=0,
        ),
    )(x)
